# baseline (speedup 1.0000x reference)
import jax
import jax.numpy as jnp
from jax import lax
from jax.experimental import pallas as pl
from jax.experimental.pallas import tpu as pltpu
from jax.experimental.pallas import tpu_sc as plsc

B = 16
T = 1984
D = 512
N = 16384
NC = 2
NS = 16
NW = NC * NS
ROWS_PW = N // NW
DCH = 64
NDC = ROWS_PW // DCH
ZCH = 64
NZC = 8
NPAD = B * T - N
VREGS_PER_ROW = D // 16


def _sc_body(sidx_hbm, zidx_hbm, data_hbm, out_hbm, sidx_v, zidx_v, rows_v,
             zeros_v, sem):
    cid = lax.axis_index("c")
    sid = lax.axis_index("s")
    wid = cid * NS + sid
    base = wid * ROWS_PW

    zvec = jnp.zeros((16,), jnp.float32)

    def zinit(r, carry):
        for j in range(VREGS_PER_ROW):
            zeros_v[r, pl.ds(j * 16, 16)] = zvec
        return carry

    lax.fori_loop(0, ZCH, zinit, 0)

    for ci in range(NDC):
        pltpu.sync_copy(sidx_hbm.at[wid, ci], sidx_v)
        pltpu.sync_copy(data_hbm.at[pl.ds(base + ci * DCH, DCH)], rows_v)
        pltpu.async_copy(rows_v, out_hbm.at[sidx_v], sem).wait()

    for ci in range(NZC):
        pltpu.sync_copy(zidx_hbm.at[wid, ci], zidx_v)
        pltpu.async_copy(zeros_v, out_hbm.at[zidx_v], sem).wait()


@jax.jit
def _pad_packed(sidx_r, zidx_r, data):
    mesh = plsc.VectorSubcoreMesh(
        core_axis_name="c", subcore_axis_name="s", num_cores=NC,
        num_subcores=NS)
    fn = pl.kernel(
        _sc_body,
        out_type=jax.ShapeDtypeStruct((B * T, D), jnp.float32),
        mesh=mesh,
        scratch_types=[
            pltpu.VMEM((DCH,), jnp.int32),
            pltpu.VMEM((ZCH,), jnp.int32),
            pltpu.VMEM((DCH, D), jnp.float32),
            pltpu.VMEM((ZCH, D), jnp.float32),
            pltpu.SemaphoreType.DMA,
        ],
    )
    return fn(sidx_r, zidx_r, data)


def kernel(data, lengths):
    lengths_i = lengths.astype(jnp.int32)
    t_range = jnp.arange(T, dtype=jnp.int32)
    batch_sizes = jnp.sum(
        (lengths_i[None, :] > t_range[:, None]).astype(jnp.int32), axis=1)
    offsets = jnp.concatenate(
        [jnp.zeros((1,), jnp.int32), jnp.cumsum(batch_sizes)[:-1]])
    idx = offsets[None, :] + jnp.arange(B, dtype=jnp.int32)[:, None]
    maskf = (t_range[None, :] < lengths_i[:, None]).reshape(-1)
    flat_bt = jnp.arange(B * T, dtype=jnp.int32)
    safe_idx = jnp.where(maskf, idx.reshape(-1), N)
    sidx = jnp.zeros((N + 1,), jnp.int32).at[safe_idx].set(flat_bt)[:N]
    zpos = jnp.where(jnp.logical_not(maskf), size=NPAD, fill_value=0)[0]
    zpos = zpos.astype(jnp.int32)
    zidx = jnp.concatenate(
        [zpos, jnp.full((NW * NZC * ZCH - NPAD,), zpos[-1], jnp.int32)])
    sidx_r = sidx.reshape(NW, NDC, DCH)
    zidx_r = zidx.reshape(NW, NZC, ZCH)
    out_flat = _pad_packed(sidx_r, zidx_r, data)
    return out_flat.reshape(B, T, D), lengths


# speedup vs baseline: 1.0516x; 1.0516x over previous
import jax
import jax.numpy as jnp
from jax import lax
from jax.experimental import pallas as pl
from jax.experimental.pallas import tpu as pltpu
from jax.experimental.pallas import tpu_sc as plsc

B = 16
T = 1984
D = 512
N = 16384
NC = 2
NS = 16
NW = NC * NS
ROWS_PW = N // NW
DCH = 64
NDC = ROWS_PW // DCH
ZCH = 64
NZC = 8
NPAD = B * T - N
VREGS_PER_ROW = D // 16


def _sc_body(sidx_hbm, zidx_hbm, data_hbm, out_hbm, sidx_v, zidx_v, rows_a,
             rows_b, zeros_v, sem_ra, sem_rb, sem_sa, sem_sb, sem_z):
    cid = lax.axis_index("c")
    sid = lax.axis_index("s")
    wid = cid * NS + sid
    base = wid * ROWS_PW

    bufs = (rows_a, rows_b)
    rsem = (sem_ra, sem_rb)
    ssem = (sem_sa, sem_sb)

    read = [None, None]
    scat = [None, None]
    read[0] = pltpu.async_copy(
        data_hbm.at[pl.ds(base, DCH)], rows_a, sem_ra)
    pltpu.sync_copy(sidx_hbm.at[wid], sidx_v)
    pltpu.sync_copy(zidx_hbm.at[wid], zidx_v)

    zvec = jnp.zeros((16,), jnp.float32)

    def zinit(r, carry):
        for j in range(VREGS_PER_ROW):
            zeros_v[r, pl.ds(j * 16, 16)] = zvec
        return carry

    lax.fori_loop(0, ZCH, zinit, 0)

    zhandles = []
    for ci in range(NDC):
        bi = ci & 1
        nb = 1 - bi
        read[bi].wait()
        if ci + 1 < NDC:
            if scat[nb] is not None:
                scat[nb].wait()
                scat[nb] = None
            read[nb] = pltpu.async_copy(
                data_hbm.at[pl.ds(base + (ci + 1) * DCH, DCH)], bufs[nb],
                rsem[nb])
        scat[bi] = pltpu.async_copy(bufs[bi], out_hbm.at[sidx_v.at[ci]],
                                    ssem[bi])
        zhandles.append(
            pltpu.async_copy(zeros_v, out_hbm.at[zidx_v.at[ci]], sem_z))

    for h in scat:
        if h is not None:
            h.wait()
    for h in zhandles:
        h.wait()


@jax.jit
def _pad_packed(sidx_r, zidx_r, data):
    mesh = plsc.VectorSubcoreMesh(
        core_axis_name="c", subcore_axis_name="s", num_cores=NC,
        num_subcores=NS)
    fn = pl.kernel(
        _sc_body,
        out_type=jax.ShapeDtypeStruct((B * T, D), jnp.float32),
        mesh=mesh,
        scratch_types=[
            pltpu.VMEM((NDC, DCH), jnp.int32),
            pltpu.VMEM((NZC, ZCH), jnp.int32),
            pltpu.VMEM((DCH, D), jnp.float32),
            pltpu.VMEM((DCH, D), jnp.float32),
            pltpu.VMEM((ZCH, D), jnp.float32),
            pltpu.SemaphoreType.DMA,
            pltpu.SemaphoreType.DMA,
            pltpu.SemaphoreType.DMA,
            pltpu.SemaphoreType.DMA,
            pltpu.SemaphoreType.DMA,
        ],
    )
    return fn(sidx_r, zidx_r, data)


def kernel(data, lengths):
    lengths_i = lengths.astype(jnp.int32)
    t_range = jnp.arange(T, dtype=jnp.int32)
    batch_sizes = jnp.sum(
        (lengths_i[None, :] > t_range[:, None]).astype(jnp.int32), axis=1)
    offsets = jnp.concatenate(
        [jnp.zeros((1,), jnp.int32), jnp.cumsum(batch_sizes)[:-1]])
    idx = offsets[None, :] + jnp.arange(B, dtype=jnp.int32)[:, None]
    maskf = (t_range[None, :] < lengths_i[:, None]).reshape(-1)
    flat_bt = jnp.arange(B * T, dtype=jnp.int32)
    safe_idx = jnp.where(maskf, idx.reshape(-1), N)
    sidx = jnp.zeros((N + 1,), jnp.int32).at[safe_idx].set(flat_bt)[:N]
    zpos = jnp.where(jnp.logical_not(maskf), size=NPAD, fill_value=0)[0]
    zpos = zpos.astype(jnp.int32)
    zidx = jnp.concatenate(
        [zpos, jnp.full((NW * NZC * ZCH - NPAD,), zpos[-1], jnp.int32)])
    sidx_r = sidx.reshape(NW, NDC, DCH)
    zidx_r = zidx.reshape(NW, NZC, ZCH)
    out_flat = _pad_packed(sidx_r, zidx_r, data)
    return out_flat.reshape(B, T, D), lengths


# speedup vs baseline: 2.3601x; 2.2443x over previous
import jax
import jax.numpy as jnp
from jax import lax
from jax.experimental import pallas as pl
from jax.experimental.pallas import tpu as pltpu
from jax.experimental.pallas import tpu_sc as plsc

B = 16
T = 1984
D = 512
N = 16384
NC = 2
NS = 16
NW = NC * NS
ROWS_PW = N // NW
DCH = 64
NDC = ROWS_PW // DCH
ZCH = 64
NZC = 8
NPAD = B * T - N
VREGS_PER_ROW = D // 16


def _sc_body(sidx_hbm, zidx_hbm, data_hbm, out_hbm, sidx_v, zidx_v, rows_a,
             rows_b, zeros_v, sem_ra, sem_rb, sem_sa, sem_sb, sem_z):
    cid = lax.axis_index("c")
    sid = lax.axis_index("s")
    wid = cid * NS + sid
    base = wid * ROWS_PW

    bufs = (rows_a, rows_b)
    rsem = (sem_ra, sem_rb)
    ssem = (sem_sa, sem_sb)

    read = [None, None]
    scat = [None, None]
    read[0] = pltpu.async_copy(
        data_hbm.at[pl.ds(base, DCH)], rows_a, sem_ra)
    pltpu.sync_copy(sidx_hbm.at[wid], sidx_v)
    pltpu.sync_copy(zidx_hbm.at[wid], zidx_v)

    zvec = jnp.zeros((16,), jnp.float32)

    def zinit(r, carry):
        for j in range(VREGS_PER_ROW):
            zeros_v[r, pl.ds(j * 16, 16)] = zvec
        return carry

    lax.fori_loop(0, ZCH, zinit, 0)

    zhandles = []
    for ci in range(NDC):
        bi = ci & 1
        nb = 1 - bi
        read[bi].wait()
        if ci + 1 < NDC:
            if scat[nb] is not None:
                scat[nb].wait()
                scat[nb] = None
            read[nb] = pltpu.async_copy(
                data_hbm.at[pl.ds(base + (ci + 1) * DCH, DCH)], bufs[nb],
                rsem[nb])
        scat[bi] = pltpu.async_copy(bufs[bi], out_hbm.at[sidx_v.at[ci]],
                                    ssem[bi])
        zhandles.append(
            pltpu.async_copy(zeros_v, out_hbm.at[zidx_v.at[ci]], sem_z))

    for h in scat:
        if h is not None:
            h.wait()
    for h in zhandles:
        h.wait()


@jax.jit
def _pad_packed(sidx_r, zidx_r, data):
    mesh = plsc.VectorSubcoreMesh(
        core_axis_name="c", subcore_axis_name="s", num_cores=NC,
        num_subcores=NS)
    fn = pl.kernel(
        _sc_body,
        out_type=jax.ShapeDtypeStruct((B * T, D), jnp.float32),
        mesh=mesh,
        scratch_types=[
            pltpu.VMEM((NDC, DCH), jnp.int32),
            pltpu.VMEM((NZC, ZCH), jnp.int32),
            pltpu.VMEM((DCH, D), jnp.float32),
            pltpu.VMEM((DCH, D), jnp.float32),
            pltpu.VMEM((ZCH, D), jnp.float32),
            pltpu.SemaphoreType.DMA,
            pltpu.SemaphoreType.DMA,
            pltpu.SemaphoreType.DMA,
            pltpu.SemaphoreType.DMA,
            pltpu.SemaphoreType.DMA,
        ],
    )
    return fn(sidx_r, zidx_r, data)


def kernel(data, lengths):
    L = lengths.astype(jnp.int32)
    karr = jnp.arange(B, dtype=jnp.int32)
    Lext = jnp.concatenate([L, jnp.zeros((1,), jnp.int32)])
    t_lo = Lext[B - karr]
    t_hi = Lext[B - karr - 1]
    seg_rows = (t_hi - t_lo) * (B - karr)
    R = jnp.concatenate(
        [jnp.zeros((1,), jnp.int32), jnp.cumsum(seg_rows)[:-1]])
    r = jnp.arange(N, dtype=jnp.int32)
    kk = jnp.sum((r[:, None] >= R[None, :]).astype(jnp.int32), axis=1) - 1
    onehot = (kk[:, None] == karr[None, :]).astype(jnp.int32)
    R_r = jnp.sum(onehot * R[None, :], axis=1)
    tlo_r = jnp.sum(onehot * t_lo[None, :], axis=1)
    s_r = B - kk
    local = r - R_r
    sidx = (local % s_r) * T + tlo_r + local // s_r
    cpad = jnp.concatenate(
        [jnp.zeros((1,), jnp.int32), jnp.cumsum(T - L)[:-1]])
    j = jnp.arange(NW * NZC * ZCH, dtype=jnp.int32)
    j = jnp.minimum(j, NPAD - 1)
    bj = jnp.sum((j[:, None] >= cpad[None, :]).astype(jnp.int32), axis=1) - 1
    oh = (bj[:, None] == karr[None, :]).astype(jnp.int32)
    cpad_j = jnp.sum(oh * cpad[None, :], axis=1)
    L_j = jnp.sum(oh * L[None, :], axis=1)
    zidx = bj * T + L_j + (j - cpad_j)
    sidx_r = sidx.reshape(NW, NDC, DCH)
    zidx_r = zidx.reshape(NW, NZC, ZCH)
    out_flat = _pad_packed(sidx_r, zidx_r, data)
    return out_flat.reshape(B, T, D), lengths
